# baseline (speedup 1.0000x reference)
import functools

import jax
import jax.numpy as jnp
from jax import lax
from jax.experimental import pallas as pl
from jax.experimental.pallas import tpu as pltpu
from jax.experimental.pallas import tpu_sc as plsc

_D = 32
_NC = 2
_NS = 16
_NW = _NC * _NS
_NB = 8
_NPH = 2


@functools.lru_cache(maxsize=None)
def _gather_call(bsz, seq):
    bpw = bsz // _NW
    hpw = bpw // _NPH
    ng = hpw // _NB
    segs = []
    off = 0
    while off < seq:
        n = min(128, seq - off)
        segs.append((off, n))
        off += n
    mesh = plsc.VectorSubcoreMesh(core_axis_name="c", subcore_axis_name="s")

    @functools.partial(
        pl.kernel,
        mesh=mesh,
        out_type=jax.ShapeDtypeStruct((bsz, seq, _D), jnp.float32),
        scratch_types=[
            pltpu.VMEM((hpw, seq), jnp.int32),
            pltpu.VMEM((2, _NB, seq, _D), jnp.float32),
            pltpu.SemaphoreType.DMA,
            (pltpu.SemaphoreType.DMA, pltpu.SemaphoreType.DMA),
        ],
        compiler_params=pltpu.CompilerParams(use_tc_tiling_on_sc=False),
    )
    def k(idx_hbm, tab_hbm, out_hbm, idx_v, rows_v, gsem, wsems):
        wid = lax.axis_index("s") * _NC + lax.axis_index("c")
        brow0 = wid * bpw

        def wb_wait(b):
            pltpu.make_async_copy(
                rows_v.at[b], out_hbm.at[pl.ds(0, _NB)], wsems[b]).wait()

        def do_chunk(base, g, b):
            copies = [
                pltpu.async_copy(
                    tab_hbm.at[idx_v.at[g * _NB + r, pl.ds(soff, slen)]],
                    rows_v.at[b, r, pl.ds(soff, slen)],
                    gsem,
                )
                for r in range(_NB)
                for soff, slen in segs
            ]
            for c in copies:
                c.wait()
            pltpu.make_async_copy(
                rows_v.at[b],
                out_hbm.at[pl.ds(base + g * _NB, _NB)],
                wsems[b],
            ).start()

        for p in range(_NPH):
            base = brow0 + p * hpw
            pltpu.sync_copy(idx_hbm.at[pl.ds(base, hpw)], idx_v)

            def body(g2, carry, base=base):
                g = g2 * 2

                @pl.when(g2 > 0)
                def _():
                    wb_wait(0)

                do_chunk(base, g, 0)

                @pl.when(g2 > 0)
                def _():
                    wb_wait(1)

                do_chunk(base, g + 1, 1)
                return carry

            lax.fori_loop(0, ng // 2, body, 0)
            wb_wait(0)
            wb_wait(1)

    return k


def kernel(x, w):
    bsz, seq = x.shape
    assert bsz % (_NW * _NPH * 2 * _NB) == 0 and seq % 8 == 0
    return _gather_call(bsz, seq)(x.astype(jnp.int32), w)
